# baseline (speedup 1.0000x reference)
import jax
import jax.numpy as jnp
from jax.experimental import pallas as pl
from jax.experimental.pallas import tpu as pltpu

N = 32768
D = 768
E = 16

_R = 128
_C = N // _R
_FB = 1024
_NB = N // _FB

_T_SCHED = E + 1
_T_COMPUTE = _NB - 2
_T_MERGE = _NB - 1

_M_COUNT = 0
_M_LAST = 16
_M_ESEL = 32
_M_VALID = 48
_M_WSEL = 64
_M_U = 80


def _main_kernel(inds_ref, x_ref, w_hbm, b_ref, out_ref,
                 rows_ref, wbuf_ref, meta_ref, sems):
    t = pl.program_id(0)

    @pl.when((t >= 1) & (t <= E))
    def _():
        j = t - 1
        inds = inds_ref[...]
        lin = (jax.lax.broadcasted_iota(jnp.int32, (_R, _C), 0) * _C
               + jax.lax.broadcasted_iota(jnp.int32, (_R, _C), 1))
        m = inds == j
        meta_ref[_M_COUNT + j] = jnp.sum(m.astype(jnp.int32))
        meta_ref[_M_LAST + j] = jnp.max(jnp.where(m, lin, -1))

    @pl.when(t == _T_SCHED)
    def _():
        counts = [meta_ref[_M_COUNT + j] for j in range(E)]
        lasts = [meta_ref[_M_LAST + j] for j in range(E)]
        cums = []
        acc = counts[0]
        cums.append(acc)
        for j in range(1, E):
            acc = acc + counts[j]
            cums.append(acc)
        e_sel = []
        valid = []
        for j in range(E):
            e = counts[0] * 0
            for k in range(E):
                e = e + (cums[k] <= lasts[j]).astype(jnp.int32)
            e_sel.append(jnp.minimum(e, E - 1))
            valid.append((counts[j] > 0).astype(jnp.int32))
        used = []
        for e in range(E):
            u = counts[0] * 0
            for j in range(E):
                u = u | (valid[j] & (e_sel[j] == e).astype(jnp.int32))
            used.append(u)
        rank = []
        r = counts[0] * 0
        for e in range(E):
            rank.append(r)
            r = r + used[e]
        num_used = r
        wsel = []
        for s in range(E):
            idx = jnp.minimum(jnp.int32(s), num_used - 1)
            w = counts[0] * 0
            for e in range(E):
                w = w + e * used[e] * (rank[e] == idx).astype(jnp.int32)
            wsel.append(w)
        for j in range(E):
            meta_ref[_M_ESEL + j] = e_sel[j]
            meta_ref[_M_VALID + j] = valid[j]
            meta_ref[_M_WSEL + j] = wsel[j]
        meta_ref[_M_U] = num_used
        for s in range(E):
            @pl.when(s < num_used)
            def _():
                pltpu.make_async_copy(
                    w_hbm.at[wsel[s]], wbuf_ref.at[s], sems.at[s]).start()

    out_ref[...] = jnp.zeros_like(out_ref)

    @pl.when(t == _T_COMPUTE)
    def _():
        rows_ref[...] = jnp.zeros_like(rows_ref)
        num_used = meta_ref[_M_U]
        for s in range(E):
            @pl.when(s < num_used)
            def _():
                cur = meta_ref[_M_WSEL + s]
                pltpu.make_async_copy(
                    w_hbm.at[cur], wbuf_ref.at[s], sems.at[s]).wait()
                y = jax.lax.dot_general(
                    x_ref[...], wbuf_ref[s], (((1,), (1,)), ((), ())),
                    preferred_element_type=jnp.float32)
                onehot = (jax.lax.broadcasted_iota(jnp.int32, (1, E), 1)
                          == cur).astype(jnp.float32)
                y = y + jax.lax.dot_general(
                    onehot, b_ref[...], (((1,), (0,)), ((), ())),
                    preferred_element_type=jnp.float32)
                for j in range(E):
                    @pl.when((meta_ref[_M_VALID + j] == 1)
                             & (meta_ref[_M_ESEL + j] == cur))
                    def _():
                        rows_ref[j:j + 1, :] = y[j:j + 1, :]

    @pl.when(t == _T_MERGE)
    def _():
        out_ref[0:E, :] = rows_ref[...]


def kernel(features, inds, W, b):
    inds2d = inds.astype(jnp.int32).reshape(_R, _C)

    out = pl.pallas_call(
        _main_kernel,
        grid=(_NB,),
        in_specs=[
            pl.BlockSpec((_R, _C), lambda t: (0, 0)),
            pl.BlockSpec((E, D), lambda t: (0, 0)),
            pl.BlockSpec(memory_space=pltpu.MemorySpace.HBM),
            pl.BlockSpec((E, D), lambda t: (0, 0)),
        ],
        out_specs=pl.BlockSpec((_FB, D), lambda t: ((t + 1) % _NB, 0)),
        out_shape=jax.ShapeDtypeStruct((N, D), jnp.float32),
        scratch_shapes=[
            pltpu.VMEM((E, D), jnp.float32),
            pltpu.VMEM((E, D, D), jnp.float32),
            pltpu.SMEM((96,), jnp.int32),
            pltpu.SemaphoreType.DMA((E,)),
        ],
    )(inds2d, features, W, b)
    return out


# speedup vs baseline: 1.0050x; 1.0050x over previous
import jax
import jax.numpy as jnp
from jax.experimental import pallas as pl
from jax.experimental.pallas import tpu as pltpu

N = 32768
D = 768
E = 16

_R = 128
_C = N // _R
_FB = 2048
_NB = N // _FB

_M_ESEL = 0
_M_VALID = 16
_M_WSEL = 32
_M_U = 48


def _main_kernel(inds_ref, x_ref, w_hbm, b_ref, out_ref,
                 rows_ref, wbuf_ref, meta_ref, sems):
    t = pl.program_id(0)

    @pl.when(t == 0)
    def _():
        rows_ref[...] = jnp.zeros_like(rows_ref)
        inds = inds_ref[...]
        lin = (jax.lax.broadcasted_iota(jnp.int32, (_R, _C), 0) * _C
               + jax.lax.broadcasted_iota(jnp.int32, (_R, _C), 1))
        counts = []
        lasts = []
        for j in range(E):
            m = inds == j
            counts.append(jnp.sum(m.astype(jnp.int32)))
            lasts.append(jnp.max(jnp.where(m, lin, -1)))
        cums = []
        acc = counts[0]
        cums.append(acc)
        for j in range(1, E):
            acc = acc + counts[j]
            cums.append(acc)
        e_sel = []
        valid = []
        for j in range(E):
            e = counts[0] * 0
            for k in range(E):
                e = e + (cums[k] <= lasts[j]).astype(jnp.int32)
            e_sel.append(jnp.minimum(e, E - 1))
            valid.append((counts[j] > 0).astype(jnp.int32))
        used = []
        for e in range(E):
            u = counts[0] * 0
            for j in range(E):
                u = u | (valid[j] & (e_sel[j] == e).astype(jnp.int32))
            used.append(u)
        rank = []
        r = counts[0] * 0
        for e in range(E):
            rank.append(r)
            r = r + used[e]
        num_used = r
        wsel = []
        for s in range(E):
            idx = jnp.minimum(jnp.int32(s), num_used - 1)
            w = counts[0] * 0
            for e in range(E):
                w = w + e * used[e] * (rank[e] == idx).astype(jnp.int32)
            wsel.append(w)
        for j in range(E):
            meta_ref[_M_ESEL + j] = e_sel[j]
            meta_ref[_M_VALID + j] = valid[j]
            meta_ref[_M_WSEL + j] = wsel[j]
        meta_ref[_M_U] = num_used
        for s in range(E):
            @pl.when(s < num_used)
            def _():
                pltpu.make_async_copy(
                    w_hbm.at[wsel[s]], wbuf_ref.at[s], sems.at[s]).start()

    out_ref[...] = jnp.zeros_like(out_ref)

    @pl.when(t == _NB - 1)
    def _():
        num_used = meta_ref[_M_U]
        for s in range(E):
            @pl.when(s < num_used)
            def _():
                cur = meta_ref[_M_WSEL + s]
                pltpu.make_async_copy(
                    w_hbm.at[cur], wbuf_ref.at[s], sems.at[s]).wait()
                y = jax.lax.dot_general(
                    x_ref[...], wbuf_ref[s], (((1,), (1,)), ((), ())),
                    preferred_element_type=jnp.float32)
                onehot = (jax.lax.broadcasted_iota(jnp.int32, (1, E), 1)
                          == cur).astype(jnp.float32)
                y = y + jax.lax.dot_general(
                    onehot, b_ref[...], (((1,), (0,)), ((), ())),
                    preferred_element_type=jnp.float32)
                for j in range(E):
                    @pl.when((meta_ref[_M_VALID + j] == 1)
                             & (meta_ref[_M_ESEL + j] == cur))
                    def _():
                        rows_ref[j:j + 1, :] = y[j:j + 1, :]
        out_ref[0:E, :] = rows_ref[...]


def kernel(features, inds, W, b):
    inds2d = inds.astype(jnp.int32).reshape(_R, _C)

    out = pl.pallas_call(
        _main_kernel,
        grid=(_NB,),
        in_specs=[
            pl.BlockSpec((_R, _C), lambda t: (0, 0)),
            pl.BlockSpec((E, D), lambda t: (0, 0)),
            pl.BlockSpec(memory_space=pltpu.MemorySpace.HBM),
            pl.BlockSpec((E, D), lambda t: (0, 0)),
        ],
        out_specs=pl.BlockSpec((_FB, D), lambda t: ((t + 1) % _NB, 0)),
        out_shape=jax.ShapeDtypeStruct((N, D), jnp.float32),
        scratch_shapes=[
            pltpu.VMEM((E, D), jnp.float32),
            pltpu.VMEM((E, D, D), jnp.float32),
            pltpu.SMEM((64,), jnp.int32),
            pltpu.SemaphoreType.DMA((E,)),
        ],
    )(inds2d, features, W, b)
    return out


# speedup vs baseline: 1.0642x; 1.0589x over previous
import jax
import jax.numpy as jnp
from jax.experimental import pallas as pl
from jax.experimental.pallas import tpu as pltpu

N = 32768
D = 768
E = 16

_R = 128
_C = N // _R
_FB = 2048
_NB = N // _FB

_M_COUNT = 0
_M_LAST = 16
_M_ESEL = 32
_M_VALID = 48
_M_WSEL = 64
_M_U = 80


def _main_kernel(inds_ref, x_ref, w_hbm, b_ref, out_ref,
                 rows_ref, wbuf_ref, meta_ref, sems):
    t = pl.program_id(0)

    def scan_half(lo):
        inds = inds_ref[...]
        lin = (jax.lax.broadcasted_iota(jnp.int32, (_R, _C), 0) * _C
               + jax.lax.broadcasted_iota(jnp.int32, (_R, _C), 1))
        for j in range(lo, lo + E // 2):
            m = inds == j
            meta_ref[_M_COUNT + j] = jnp.sum(m.astype(jnp.int32))
            meta_ref[_M_LAST + j] = jnp.max(jnp.where(m, lin, -1))

    @pl.when(t == 0)
    def _():
        rows_ref[...] = jnp.zeros_like(rows_ref)
        scan_half(0)

    @pl.when(t == 1)
    def _():
        scan_half(E // 2)
        counts = [meta_ref[_M_COUNT + j] for j in range(E)]
        lasts = [meta_ref[_M_LAST + j] for j in range(E)]
        cums = []
        acc = counts[0]
        cums.append(acc)
        for j in range(1, E):
            acc = acc + counts[j]
            cums.append(acc)
        e_sel = []
        valid = []
        for j in range(E):
            e = counts[0] * 0
            for k in range(E):
                e = e + (cums[k] <= lasts[j]).astype(jnp.int32)
            e_sel.append(jnp.minimum(e, E - 1))
            valid.append((counts[j] > 0).astype(jnp.int32))
        used = []
        for e in range(E):
            u = counts[0] * 0
            for j in range(E):
                u = u | (valid[j] & (e_sel[j] == e).astype(jnp.int32))
            used.append(u)
        rank = []
        r = counts[0] * 0
        for e in range(E):
            rank.append(r)
            r = r + used[e]
        num_used = r
        wsel = []
        for s in range(E):
            idx = jnp.minimum(jnp.int32(s), num_used - 1)
            w = counts[0] * 0
            for e in range(E):
                w = w + e * used[e] * (rank[e] == idx).astype(jnp.int32)
            wsel.append(w)
        for j in range(E):
            meta_ref[_M_ESEL + j] = e_sel[j]
            meta_ref[_M_VALID + j] = valid[j]
            meta_ref[_M_WSEL + j] = wsel[j]
        meta_ref[_M_U] = num_used
        for s in range(E):
            @pl.when(s < num_used)
            def _():
                pltpu.make_async_copy(
                    w_hbm.at[wsel[s]], wbuf_ref.at[s], sems.at[s]).start()

    out_ref[...] = jnp.zeros_like(out_ref)

    @pl.when(t == _NB - 1)
    def _():
        num_used = meta_ref[_M_U]
        for s in range(E):
            @pl.when(s < num_used)
            def _():
                cur = meta_ref[_M_WSEL + s]
                pltpu.make_async_copy(
                    w_hbm.at[cur], wbuf_ref.at[s], sems.at[s]).wait()
                y = jax.lax.dot_general(
                    x_ref[...], wbuf_ref[s], (((1,), (1,)), ((), ())),
                    preferred_element_type=jnp.float32)
                onehot = (jax.lax.broadcasted_iota(jnp.int32, (1, E), 1)
                          == cur).astype(jnp.float32)
                y = y + jax.lax.dot_general(
                    onehot, b_ref[...], (((1,), (0,)), ((), ())),
                    preferred_element_type=jnp.float32)
                for j in range(E):
                    @pl.when((meta_ref[_M_VALID + j] == 1)
                             & (meta_ref[_M_ESEL + j] == cur))
                    def _():
                        rows_ref[j:j + 1, :] = y[j:j + 1, :]
        out_ref[0:E, :] = rows_ref[...]


def kernel(features, inds, W, b):
    inds2d = inds.astype(jnp.int32).reshape(_R, _C)

    out = pl.pallas_call(
        _main_kernel,
        grid=(_NB,),
        in_specs=[
            pl.BlockSpec((_R, _C), lambda t: (0, 0)),
            pl.BlockSpec((E, D), lambda t: (0, 0)),
            pl.BlockSpec(memory_space=pltpu.MemorySpace.HBM),
            pl.BlockSpec((E, D), lambda t: (0, 0)),
        ],
        out_specs=pl.BlockSpec((_FB, D), lambda t: ((t + 1) % _NB, 0)),
        out_shape=jax.ShapeDtypeStruct((N, D), jnp.float32),
        scratch_shapes=[
            pltpu.VMEM((E, D), jnp.float32),
            pltpu.VMEM((E, D, D), jnp.float32),
            pltpu.SMEM((96,), jnp.int32),
            pltpu.SemaphoreType.DMA((E,)),
        ],
    )(inds2d, features, W, b)
    return out


# speedup vs baseline: 1.0650x; 1.0007x over previous
import jax
import jax.numpy as jnp
from jax.experimental import pallas as pl
from jax.experimental.pallas import tpu as pltpu

N = 32768
D = 768
E = 16

_R = 128
_C = N // _R
_FB = 2048
_NB = N // _FB

_M_COUNT = 0
_M_LAST = 16
_M_ESEL = 32
_M_VALID = 48
_M_WSEL = 64
_M_U = 80


def _main_kernel(inds_ref, x_ref, w_hbm, b_ref, out_ref,
                 rows_ref, wbuf_ref, meta_ref, sems):
    t = pl.program_id(0)

    def scan_quarter(lo):
        inds = inds_ref[...]
        lin = (jax.lax.broadcasted_iota(jnp.int32, (_R, _C), 0) * _C
               + jax.lax.broadcasted_iota(jnp.int32, (_R, _C), 1))
        for j in range(lo, lo + E // 4):
            m = inds == j
            meta_ref[_M_COUNT + j] = jnp.sum(m.astype(jnp.int32))
            meta_ref[_M_LAST + j] = jnp.max(jnp.where(m, lin, -1))

    @pl.when(t == 0)
    def _():
        rows_ref[...] = jnp.zeros_like(rows_ref)
        scan_quarter(0)

    for _q in range(1, 4):
        @pl.when(t == _q)
        def _(_lo=_q * (E // 4)):
            scan_quarter(_lo)

    @pl.when(t == 4)
    def _():
        counts = [meta_ref[_M_COUNT + j] for j in range(E)]
        lasts = [meta_ref[_M_LAST + j] for j in range(E)]
        cums = []
        acc = counts[0]
        cums.append(acc)
        for j in range(1, E):
            acc = acc + counts[j]
            cums.append(acc)
        e_sel = []
        valid = []
        for j in range(E):
            e = counts[0] * 0
            for k in range(E):
                e = e + (cums[k] <= lasts[j]).astype(jnp.int32)
            e_sel.append(jnp.minimum(e, E - 1))
            valid.append((counts[j] > 0).astype(jnp.int32))
        used = []
        for e in range(E):
            u = counts[0] * 0
            for j in range(E):
                u = u | (valid[j] & (e_sel[j] == e).astype(jnp.int32))
            used.append(u)
        rank = []
        r = counts[0] * 0
        for e in range(E):
            rank.append(r)
            r = r + used[e]
        num_used = r
        wsel = []
        for s in range(E):
            idx = jnp.minimum(jnp.int32(s), num_used - 1)
            w = counts[0] * 0
            for e in range(E):
                w = w + e * used[e] * (rank[e] == idx).astype(jnp.int32)
            wsel.append(w)
        for j in range(E):
            meta_ref[_M_ESEL + j] = e_sel[j]
            meta_ref[_M_VALID + j] = valid[j]
            meta_ref[_M_WSEL + j] = wsel[j]
        meta_ref[_M_U] = num_used
        for s in range(E):
            @pl.when(s < num_used)
            def _():
                pltpu.make_async_copy(
                    w_hbm.at[wsel[s]], wbuf_ref.at[s], sems.at[s]).start()

    out_ref[...] = jnp.zeros_like(out_ref)

    @pl.when(t == _NB - 2)
    def _():
        num_used = meta_ref[_M_U]
        for s in range(E):
            @pl.when(s < num_used)
            def _():
                cur = meta_ref[_M_WSEL + s]
                pltpu.make_async_copy(
                    w_hbm.at[cur], wbuf_ref.at[s], sems.at[s]).wait()
                y = jax.lax.dot_general(
                    x_ref[...], wbuf_ref[s], (((1,), (1,)), ((), ())),
                    preferred_element_type=jnp.float32)
                onehot = (jax.lax.broadcasted_iota(jnp.int32, (1, E), 1)
                          == cur).astype(jnp.float32)
                y = y + jax.lax.dot_general(
                    onehot, b_ref[...], (((1,), (0,)), ((), ())),
                    preferred_element_type=jnp.float32)
                for j in range(E):
                    @pl.when((meta_ref[_M_VALID + j] == 1)
                             & (meta_ref[_M_ESEL + j] == cur))
                    def _():
                        rows_ref[j:j + 1, :] = y[j:j + 1, :]

    @pl.when(t == _NB - 1)
    def _():
        out_ref[0:E, :] = rows_ref[...]


def kernel(features, inds, W, b):
    inds2d = inds.astype(jnp.int32).reshape(_R, _C)

    out = pl.pallas_call(
        _main_kernel,
        grid=(_NB,),
        in_specs=[
            pl.BlockSpec((_R, _C), lambda t: (0, 0)),
            pl.BlockSpec((E, D), lambda t: (0, 0)),
            pl.BlockSpec(memory_space=pltpu.MemorySpace.HBM),
            pl.BlockSpec((E, D), lambda t: (0, 0)),
        ],
        out_specs=pl.BlockSpec((_FB, D), lambda t: ((t + 1) % _NB, 0)),
        out_shape=jax.ShapeDtypeStruct((N, D), jnp.float32),
        scratch_shapes=[
            pltpu.VMEM((E, D), jnp.float32),
            pltpu.VMEM((E, D, D), jnp.float32),
            pltpu.SMEM((96,), jnp.int32),
            pltpu.SemaphoreType.DMA((E,)),
        ],
    )(inds2d, features, W, b)
    return out
